# baseline (speedup 1.0000x reference)
import functools

import jax
import jax.numpy as jnp
from jax import lax
from jax.experimental import pallas as pl
from jax.experimental.pallas import tpu as pltpu
from jax.experimental.pallas import tpu_sc as plsc

_Q = 2048
_K = 2048
_VROW = 4096
_HALF = _Q // 2
_NHEADS = 16
_QB = 8
_KB = 128
_NKT = _K // _KB
_WIN = 4


def _sc_body(table_hbm, d0_hbm, out_hbm, table_v, d0_v, v_ref, stage, sem):
    (tbl_n,) = table_v.shape
    nheads = _NHEADS
    max_d = (tbl_n // nheads - 1) // 2

    wid = lax.axis_index("s") * 2 + lax.axis_index("c")
    h = wid // 2
    q0 = (wid % 2) * _HALF
    qb0 = q0 // _QB

    pltpu.sync_copy(table_hbm, table_v)
    pltpu.sync_copy(d0_hbm, d0_v)
    d0vec = d0_v[...]
    d0s = jnp.max(d0vec)

    lanes = lax.iota(jnp.int32, 16)
    col16 = jnp.full((16,), h, dtype=jnp.int32)

    def build_v(c, carry):
        j = 16 * c + lanes
        d = j - (_Q - 1) + d0vec
        ridx = jnp.clip(d, -max_d, max_d) + max_d
        v_ref[pl.ds(16 * c, 16)] = plsc.load_gather(table_v, [ridx * nheads + h])
        return carry

    lax.fori_loop(0, _VROW // 16, build_v, 0)

    clo = plsc.load_gather(table_v, [col16])
    chi = plsc.load_gather(table_v, [2 * max_d * nheads + col16])

    for b in range(2):
        def init_hi(i, carry, b=b):
            kb = i // _QB
            qi = i % _QB
            for t in range(_KB // 16):
                stage[b, kb, qi, pl.ds(16 * t, 16)] = chi
            return carry

        lax.fori_loop(0, _NKT * _QB, init_hi, 0)

    def outer(jj, carry):
        tvp = carry

        @pl.when(jj > 0)
        def _drain():
            pltpu.make_async_copy(stage.at[0], out_hbm.at[0, 0], sem).wait()
            pltpu.make_async_copy(stage.at[0], out_hbm.at[0, 0], sem).wait()

        newtv = []
        for b in range(2):
            qbg = qb0 + 2 * jj + b
            m = _QB * qbg - (max_d - 1) - d0s
            tv = jnp.clip(
                lax.shift_right_arithmetic(m, 7), 0, _NKT - _WIN
            )

            def refill(kb, carry2, b=b):
                for qi in range(_QB):
                    for t in range(_KB // 16):
                        stage[b, kb, qi, pl.ds(16 * t, 16)] = clo
                return carry2

            lax.fori_loop(tvp[b], tv, refill, 0)

            for kbp in range(_WIN):
                kb = tv + kbp
                for qi in range(_QB):
                    o = (_Q - 1) - (_QB * qbg + qi)
                    base = o + _KB * kb
                    for t in range(_KB // 16):
                        stage[b, kb, qi, pl.ds(16 * t, 16)] = v_ref[
                            pl.ds(base + 16 * t, 16)
                        ]

            pltpu.async_copy(stage.at[b], out_hbm.at[h, qbg], sem)
            newtv.append(tv)

        return tuple(newtv)

    zero = jnp.int32(0)
    lax.fori_loop(0, _HALF // (2 * _QB), outer, (zero, zero))
    pltpu.make_async_copy(stage.at[0], out_hbm.at[0, 0], sem).wait()
    pltpu.make_async_copy(stage.at[0], out_hbm.at[0, 0], sem).wait()


def kernel(qlen, klen, relative_bias_table):
    nrows, nheads = relative_bias_table.shape
    assert nheads == _NHEADS
    d0_arr = jnp.full((16,), klen - qlen, dtype=jnp.int32)

    run = functools.partial(
        pl.kernel,
        mesh=plsc.VectorSubcoreMesh(core_axis_name="c", subcore_axis_name="s"),
        compiler_params=pltpu.CompilerParams(
            needs_layout_passes=False,
            use_tc_tiling_on_sc=False,
        ),
        out_type=jax.ShapeDtypeStruct(
            (nheads, _Q // _QB, _NKT, _QB, _KB), jnp.float32
        ),
        scratch_types=[
            pltpu.VMEM((nrows * nheads,), jnp.float32),
            pltpu.VMEM((16,), jnp.int32),
            pltpu.VMEM((_VROW,), jnp.float32),
            pltpu.VMEM((2, _NKT, _QB, _KB), jnp.float32),
            pltpu.SemaphoreType.DMA,
        ],
    )(_sc_body)

    out5 = run(relative_bias_table.reshape(-1), d0_arr)
    return jnp.transpose(out5, (0, 1, 3, 2, 4)).reshape(nheads, _Q, _K)


# speedup vs baseline: 1.2821x; 1.2821x over previous
import functools

import jax
import jax.numpy as jnp
from jax import lax
from jax.experimental import pallas as pl
from jax.experimental.pallas import tpu as pltpu
from jax.experimental.pallas import tpu_sc as plsc

_Q = 2048
_K = 2048
_VROW = 4096
_HALF = _Q // 2
_NHEADS = 16
_QB = 8
_KB = 128
_NKT = _K // _KB
_WIN = 4
_NBUF = 4


def _sc_body(table_hbm, d0_hbm, out_hbm, table_v, d0_v, v_ref, stage, sem):
    (tbl_n,) = table_v.shape
    nheads = _NHEADS
    max_d = (tbl_n // nheads - 1) // 2

    wid = lax.axis_index("s") * 2 + lax.axis_index("c")
    h = wid // 2
    q0 = (wid % 2) * _HALF
    qb0 = q0 // _QB

    pltpu.sync_copy(table_hbm, table_v)
    pltpu.sync_copy(d0_hbm, d0_v)
    d0vec = d0_v[...]
    d0s = jnp.max(d0vec)

    lanes = lax.iota(jnp.int32, 16)
    col16 = jnp.full((16,), h, dtype=jnp.int32)

    def build_v(c, carry):
        j = 16 * c + lanes
        d = j - (_Q - 1) + d0vec
        ridx = jnp.clip(d, -max_d, max_d) + max_d
        v_ref[pl.ds(16 * c, 16)] = plsc.load_gather(table_v, [ridx * nheads + h])
        return carry

    lax.fori_loop(0, _VROW // 16, build_v, 0)

    clo = plsc.load_gather(table_v, [col16])
    chi = plsc.load_gather(table_v, [2 * max_d * nheads + col16])

    def init_hi(i, carry):
        b = i // (_NKT * _QB)
        kb = (i // _QB) % _NKT
        qi = i % _QB
        for t in range(_KB // 16):
            stage[b, kb, qi, pl.ds(16 * t, 16)] = chi
        return carry

    lax.fori_loop(0, _NBUF * _NKT * _QB, init_hi, 0)

    def tv_of(qbg):
        m = _QB * qbg - (max_d - 1) - d0s
        return jnp.clip(lax.shift_right_arithmetic(m, 7), 0, _NKT - _WIN)

    def outer(jj, carry):
        b = lax.rem(jj, _NBUF)
        qbg = qb0 + jj
        tv = tv_of(qbg)
        tvp = jnp.where(jj >= _NBUF, tv_of(qbg - _NBUF), 0)

        @pl.when(jj >= _NBUF)
        def _drain():
            pltpu.make_async_copy(stage.at[0], out_hbm.at[0, 0], sem).wait()

        def refill(kb, carry2):
            for qi in range(_QB):
                for t in range(_KB // 16):
                    stage[b, kb, qi, pl.ds(16 * t, 16)] = clo
            return carry2

        lax.fori_loop(tvp, tv, refill, 0)

        for kbp in range(_WIN):
            kb = tv + kbp
            for qi in range(_QB):
                o = (_Q - 1) - (_QB * qbg + qi)
                base = o + _KB * kb
                for t in range(_KB // 16):
                    stage[b, kb, qi, pl.ds(16 * t, 16)] = v_ref[
                        pl.ds(base + 16 * t, 16)
                    ]

        pltpu.async_copy(stage.at[b], out_hbm.at[h, qbg], sem)
        return carry

    lax.fori_loop(0, _HALF // _QB, outer, 0)
    for _ in range(_NBUF):
        pltpu.make_async_copy(stage.at[0], out_hbm.at[0, 0], sem).wait()


def kernel(qlen, klen, relative_bias_table):
    nrows, nheads = relative_bias_table.shape
    assert nheads == _NHEADS
    d0_arr = jnp.full((16,), klen - qlen, dtype=jnp.int32)

    run = functools.partial(
        pl.kernel,
        mesh=plsc.VectorSubcoreMesh(core_axis_name="c", subcore_axis_name="s"),
        compiler_params=pltpu.CompilerParams(
            needs_layout_passes=False,
            use_tc_tiling_on_sc=False,
        ),
        out_type=jax.ShapeDtypeStruct(
            (nheads, _Q // _QB, _NKT, _QB, _KB), jnp.float32
        ),
        scratch_types=[
            pltpu.VMEM((nrows * nheads,), jnp.float32),
            pltpu.VMEM((16,), jnp.int32),
            pltpu.VMEM((_VROW,), jnp.float32),
            pltpu.VMEM((_NBUF, _NKT, _QB, _KB), jnp.float32),
            pltpu.SemaphoreType.DMA,
        ],
    )(_sc_body)

    out5 = run(relative_bias_table.reshape(-1), d0_arr)
    return jnp.transpose(out5, (0, 1, 3, 2, 4)).reshape(nheads, _Q, _K)
